# baseline (speedup 1.0000x reference)
import functools

import jax
import jax.numpy as jnp
import numpy as np
from jax import lax
from jax.experimental import pallas as pl
from jax.experimental.pallas import tpu as pltpu
from jax.experimental.pallas import tpu_sc as plsc

N = 10000
DEG = 32
F = 128
GW = 4 * F

NC = 5
CN = N // NC
BC = CN * DEG

NB = 2000
CNT = CN // NB

NW = 32
RPW = BC // NW
CH = 400
NCHUNK = RPW // CH


def _sc_gather(table, idx_flat):
    mesh = plsc.VectorSubcoreMesh(core_axis_name="c", subcore_axis_name="s")

    @functools.partial(
        pl.kernel,
        mesh=mesh,
        out_type=jax.ShapeDtypeStruct((BC, F), jnp.float32),
        scratch_types=[
            pltpu.VMEM((CH,), jnp.int32),
            pltpu.VMEM((CH, F), jnp.float32),
            pltpu.SemaphoreType.DMA,
        ],
    )
    def gather_kernel(table_hbm, idx_hbm, out_hbm, idx_v, rows_v, sem):
        wid = lax.axis_index("s") * 2 + lax.axis_index("c")
        base = wid * RPW

        def body(k, carry):
            off = pl.multiple_of(base + k * CH, 8)
            pltpu.sync_copy(idx_hbm.at[pl.ds(off, CH)], idx_v)
            pltpu.async_copy(table_hbm.at[idx_v], rows_v, sem).wait()
            pltpu.sync_copy(rows_v, out_hbm.at[pl.ds(off, CH)])
            return carry

        lax.fori_loop(0, NCHUNK, body, 0)

    return gather_kernel(table, idx_flat)


def _sigmoid(x):
    return 0.5 * jnp.tanh(0.5 * x) + 0.5


def _lstm_step(x, h, c, wih_ref, whh_ref, b_ref):
    gates = (
        jnp.dot(x.astype(jnp.bfloat16), wih_ref[...],
                preferred_element_type=jnp.float32)
        + jnp.dot(h, whh_ref[...], preferred_element_type=jnp.float32)
        + b_ref[...]
    )
    gs = [gates[:, k * F:(k + 1) * F] for k in range(4)]
    i = _sigmoid(gs[0])
    f = _sigmoid(gs[1])
    g = jnp.tanh(gs[2])
    o = _sigmoid(gs[3])
    c2 = f * c + i * g
    h2 = o * jnp.tanh(c2)
    return h2, c2


def _layer1_body(g_ref, feat_ref, wih_ref, whh_ref, wself_ref, wneigh_ref,
                 b_ref, bo_ref, out_ref, h_ref, c_ref):
    t = pl.program_id(1)

    @pl.when(t == 0)
    def _():
        h_ref[...] = jnp.zeros_like(h_ref)
        c_ref[...] = jnp.zeros_like(c_ref)

    h2, c2 = _lstm_step(g_ref[0], h_ref[...], c_ref[...], wih_ref, whh_ref, b_ref)
    h_ref[...] = h2.astype(jnp.bfloat16)
    c_ref[...] = c2

    @pl.when(t == DEG - 1)
    def _():
        out = (
            jnp.dot(feat_ref[...].astype(jnp.bfloat16), wself_ref[...],
                    preferred_element_type=jnp.float32)
            + jnp.dot(h2.astype(jnp.bfloat16), wneigh_ref[...],
                      preferred_element_type=jnp.float32)
            + bo_ref[...]
        )
        out_ref[...] = jnp.maximum(out, 0.0)


def _layer1(g, feat, wih_t, whh_t, wself_t, wneigh_t, bvec, bout):
    return pl.pallas_call(
        _layer1_body,
        grid=(CNT, DEG),
        in_specs=[
            pl.BlockSpec((1, NB, F), lambda i, t: (t, i, 0)),
            pl.BlockSpec((NB, F), lambda i, t: (i, 0)),
            pl.BlockSpec((F, GW), lambda i, t: (0, 0)),
            pl.BlockSpec((F, GW), lambda i, t: (0, 0)),
            pl.BlockSpec((F, F), lambda i, t: (0, 0)),
            pl.BlockSpec((F, F), lambda i, t: (0, 0)),
            pl.BlockSpec((1, GW), lambda i, t: (0, 0)),
            pl.BlockSpec((1, F), lambda i, t: (0, 0)),
        ],
        out_specs=pl.BlockSpec((NB, F), lambda i, t: (i, 0)),
        out_shape=jax.ShapeDtypeStruct((CN, F), jnp.float32),
        scratch_shapes=[
            pltpu.VMEM((NB, F), jnp.bfloat16),
            pltpu.VMEM((NB, F), jnp.float32),
        ],
    )(g, feat, wih_t, whh_t, wself_t, wneigh_t, bvec, bout)


def _layer2_body(g_ref, feat_ref, wih_ref, whh_ref, wself_ref, wneigh_ref,
                 b_ref, bo_ref, sum_ref, h_ref, c_ref, acc_ref):
    i_ = pl.program_id(0)
    t = pl.program_id(1)

    @pl.when(jnp.logical_and(i_ == 0, t == 0))
    def _():
        acc_ref[...] = jnp.zeros_like(acc_ref)

    @pl.when(t == 0)
    def _():
        h_ref[...] = jnp.zeros_like(h_ref)
        c_ref[...] = jnp.zeros_like(c_ref)

    h2, c2 = _lstm_step(g_ref[0], h_ref[...], c_ref[...], wih_ref, whh_ref, b_ref)
    h_ref[...] = h2.astype(jnp.bfloat16)
    c_ref[...] = c2

    @pl.when(t == DEG - 1)
    def _():
        out = (
            jnp.dot(feat_ref[...].astype(jnp.bfloat16), wself_ref[...],
                    preferred_element_type=jnp.float32)
            + jnp.dot(h2.astype(jnp.bfloat16), wneigh_ref[...],
                      preferred_element_type=jnp.float32)
            + bo_ref[...]
        )
        acc_ref[...] += jnp.sum(out, axis=0, keepdims=True)

    @pl.when(jnp.logical_and(i_ == CNT - 1, t == DEG - 1))
    def _():
        sum_ref[...] = acc_ref[...]


def _layer2(g, feat, wih_t, whh_t, wself_t, wneigh_t, bvec, bout):
    return pl.pallas_call(
        _layer2_body,
        grid=(CNT, DEG),
        in_specs=[
            pl.BlockSpec((1, NB, F), lambda i, t: (t, i, 0)),
            pl.BlockSpec((NB, F), lambda i, t: (i, 0)),
            pl.BlockSpec((F, GW), lambda i, t: (0, 0)),
            pl.BlockSpec((F, GW), lambda i, t: (0, 0)),
            pl.BlockSpec((F, F), lambda i, t: (0, 0)),
            pl.BlockSpec((F, F), lambda i, t: (0, 0)),
            pl.BlockSpec((1, GW), lambda i, t: (0, 0)),
            pl.BlockSpec((1, F), lambda i, t: (0, 0)),
        ],
        out_specs=pl.BlockSpec((1, F), lambda i, t: (0, 0)),
        out_shape=jax.ShapeDtypeStruct((1, F), jnp.float32),
        scratch_shapes=[
            pltpu.VMEM((NB, F), jnp.bfloat16),
            pltpu.VMEM((NB, F), jnp.float32),
            pltpu.VMEM((1, F), jnp.float32),
        ],
    )(g, feat, wih_t, whh_t, wself_t, wneigh_t, bvec, bout)


def _heads_body(parts_ref, muw_ref, mub_ref, sgw_ref, sgb_ref, mu_ref, sg_ref):
    x = jnp.sum(parts_ref[...], axis=0, keepdims=True) * (1.0 / N)
    mu_ref[...] = (
        jnp.dot(x, muw_ref[...], preferred_element_type=jnp.float32)
        + mub_ref[...]
    )
    sg_ref[...] = (
        jnp.dot(x, sgw_ref[...], preferred_element_type=jnp.float32)
        + sgb_ref[...]
    )


def _heads(parts, muw_t, mub, sgw_t, sgb):
    rep = muw_t.shape[1]
    return pl.pallas_call(
        _heads_body,
        out_shape=[
            jax.ShapeDtypeStruct((1, rep), jnp.float32),
            jax.ShapeDtypeStruct((1, rep), jnp.float32),
        ],
    )(parts, muw_t, mub, sgw_t, sgb)


def kernel(in_feat, edge_src, lstm1_Wih, lstm1_Whh, lstm1_bih, lstm1_bhh,
           fc_self1, fc_neigh1, bias1, lstm2_Wih, lstm2_Whh, lstm2_bih,
           lstm2_bhh, fc_self2, fc_neigh2, bias2, mu_W, mu_b, sigma_W, sigma_b):
    idxc = [edge_src[c * CN:(c + 1) * CN].T.reshape(-1) for c in range(NC)]
    featc = [in_feat[c * CN:(c + 1) * CN] for c in range(NC)]

    bf = jnp.bfloat16
    w1 = (lstm1_Wih.T.astype(bf), lstm1_Whh.T.astype(bf),
          fc_self1.T.astype(bf), fc_neigh1.T.astype(bf),
          (lstm1_bih + lstm1_bhh).reshape(1, GW), bias1.reshape(1, F))
    w2 = (lstm2_Wih.T.astype(bf), lstm2_Whh.T.astype(bf),
          fc_self2.T.astype(bf), fc_neigh2.T.astype(bf),
          (lstm2_bih + lstm2_bhh).reshape(1, GW), bias2.reshape(1, F))

    g1 = [None] * NC
    g1[0] = _sc_gather(in_feat, idxc[0])
    outs1 = []
    for c in range(NC):
        if c + 1 < NC:
            g1[c + 1] = _sc_gather(in_feat, idxc[c + 1])
        outs1.append(_layer1(g1[c].reshape(DEG, CN, F), featc[c], *w1))
    out1 = jnp.concatenate(outs1, axis=0)

    g2 = [None] * NC
    g2[0] = _sc_gather(out1, idxc[0])
    parts = []
    for c in range(NC):
        if c + 1 < NC:
            g2[c + 1] = _sc_gather(out1, idxc[c + 1])
        parts.append(_layer2(g2[c].reshape(DEG, CN, F),
                             outs1[c], *w2))
    mu, sigma = _heads(jnp.concatenate(parts, axis=0),
                       mu_W.T, mu_b.reshape(1, -1),
                       sigma_W.T, sigma_b.reshape(1, -1))
    return (mu, sigma)


# speedup vs baseline: 1.2290x; 1.2290x over previous
import functools

import jax
import jax.numpy as jnp
import numpy as np
from jax import lax
from jax.experimental import pallas as pl
from jax.experimental.pallas import tpu as pltpu
from jax.experimental.pallas import tpu_sc as plsc

N = 10000
DEG = 32
F = 128
GW = 4 * F

NC = 5
CN = N // NC
BC = CN * DEG

NB = 2000
CNT = CN // NB

NW = 32
RPW = BC // NW
CH = 400
NCHUNK = RPW // CH


def _sc_gather(table, idx_flat):
    mesh = plsc.VectorSubcoreMesh(core_axis_name="c", subcore_axis_name="s")

    @functools.partial(
        pl.kernel,
        mesh=mesh,
        out_type=jax.ShapeDtypeStruct((BC, F), jnp.float32),
        scratch_types=[
            pltpu.VMEM((CH,), jnp.int32),
            pltpu.VMEM((CH, F), jnp.float32),
            pltpu.SemaphoreType.DMA,
        ],
    )
    def gather_kernel(table_hbm, idx_hbm, out_hbm, idx_v, rows_v, sem):
        wid = lax.axis_index("s") * 2 + lax.axis_index("c")
        base = wid * RPW

        def body(k, carry):
            off = pl.multiple_of(base + k * CH, 8)
            pltpu.sync_copy(idx_hbm.at[pl.ds(off, CH)], idx_v)
            pltpu.async_copy(table_hbm.at[idx_v], rows_v, sem).wait()
            pltpu.sync_copy(rows_v, out_hbm.at[pl.ds(off, CH)])
            return carry

        lax.fori_loop(0, NCHUNK, body, 0)

    return gather_kernel(table, idx_flat)


def _sigmoid(x):
    return 0.5 * jnp.tanh(0.5 * x) + 0.5


def _lstm_step(xh_ref, c, wcat_ref, b_ref):
    gates = (
        jnp.dot(xh_ref[...], wcat_ref[...], preferred_element_type=jnp.float32)
        + b_ref[...]
    )
    gs = [gates[:, k * F:(k + 1) * F] for k in range(4)]
    i = _sigmoid(gs[0])
    f = _sigmoid(gs[1])
    g = jnp.tanh(gs[2])
    o = _sigmoid(gs[3])
    c2 = f * c + i * g
    h2 = o * jnp.tanh(c2)
    return h2, c2


def _layer1_body(g_ref, feat_ref, wcat_ref, wself_ref, wneigh_ref,
                 b_ref, bo_ref, out_ref, xh_ref, c_ref):
    t = pl.program_id(1)

    @pl.when(t == 0)
    def _():
        xh_ref[:, F:] = jnp.zeros((NB, F), jnp.float32)
        c_ref[...] = jnp.zeros_like(c_ref)

    xh_ref[:, :F] = g_ref[0]
    h2, c2 = _lstm_step(xh_ref, c_ref[...], wcat_ref, b_ref)
    xh_ref[:, F:] = h2
    c_ref[...] = c2

    @pl.when(t == DEG - 1)
    def _():
        out = (
            jnp.dot(feat_ref[...], wself_ref[...],
                    preferred_element_type=jnp.float32)
            + jnp.dot(h2, wneigh_ref[...],
                      preferred_element_type=jnp.float32)
            + bo_ref[...]
        )
        out_ref[...] = jnp.maximum(out, 0.0)


def _layer1(g, feat, wcat, wself_t, wneigh_t, bvec, bout):
    return pl.pallas_call(
        _layer1_body,
        grid=(CNT, DEG),
        in_specs=[
            pl.BlockSpec((1, NB, F), lambda i, t: (t, i, 0)),
            pl.BlockSpec((NB, F), lambda i, t: (i, 0)),
            pl.BlockSpec((2 * F, GW), lambda i, t: (0, 0)),
            pl.BlockSpec((F, F), lambda i, t: (0, 0)),
            pl.BlockSpec((F, F), lambda i, t: (0, 0)),
            pl.BlockSpec((1, GW), lambda i, t: (0, 0)),
            pl.BlockSpec((1, F), lambda i, t: (0, 0)),
        ],
        out_specs=pl.BlockSpec((NB, F), lambda i, t: (i, 0)),
        out_shape=jax.ShapeDtypeStruct((CN, F), jnp.float32),
        scratch_shapes=[
            pltpu.VMEM((NB, 2 * F), jnp.float32),
            pltpu.VMEM((NB, F), jnp.float32),
        ],
    )(g, feat, wcat, wself_t, wneigh_t, bvec, bout)


def _layer2_body(g_ref, feat_ref, wcat_ref, wself_ref, wneigh_ref,
                 b_ref, bo_ref, sum_ref, xh_ref, c_ref, acc_ref):
    i_ = pl.program_id(0)
    t = pl.program_id(1)

    @pl.when(jnp.logical_and(i_ == 0, t == 0))
    def _():
        acc_ref[...] = jnp.zeros_like(acc_ref)

    @pl.when(t == 0)
    def _():
        xh_ref[:, F:] = jnp.zeros((NB, F), jnp.float32)
        c_ref[...] = jnp.zeros_like(c_ref)

    xh_ref[:, :F] = g_ref[0]
    h2, c2 = _lstm_step(xh_ref, c_ref[...], wcat_ref, b_ref)
    xh_ref[:, F:] = h2
    c_ref[...] = c2

    @pl.when(t == DEG - 1)
    def _():
        out = (
            jnp.dot(feat_ref[...], wself_ref[...],
                    preferred_element_type=jnp.float32)
            + jnp.dot(h2, wneigh_ref[...],
                      preferred_element_type=jnp.float32)
            + bo_ref[...]
        )
        acc_ref[...] += jnp.sum(out, axis=0, keepdims=True)

    @pl.when(jnp.logical_and(i_ == CNT - 1, t == DEG - 1))
    def _():
        sum_ref[...] = acc_ref[...]


def _layer2(g, feat, wcat, wself_t, wneigh_t, bvec, bout):
    return pl.pallas_call(
        _layer2_body,
        grid=(CNT, DEG),
        in_specs=[
            pl.BlockSpec((1, NB, F), lambda i, t: (t, i, 0)),
            pl.BlockSpec((NB, F), lambda i, t: (i, 0)),
            pl.BlockSpec((2 * F, GW), lambda i, t: (0, 0)),
            pl.BlockSpec((F, F), lambda i, t: (0, 0)),
            pl.BlockSpec((F, F), lambda i, t: (0, 0)),
            pl.BlockSpec((1, GW), lambda i, t: (0, 0)),
            pl.BlockSpec((1, F), lambda i, t: (0, 0)),
        ],
        out_specs=pl.BlockSpec((1, F), lambda i, t: (0, 0)),
        out_shape=jax.ShapeDtypeStruct((1, F), jnp.float32),
        scratch_shapes=[
            pltpu.VMEM((NB, 2 * F), jnp.float32),
            pltpu.VMEM((NB, F), jnp.float32),
            pltpu.VMEM((1, F), jnp.float32),
        ],
    )(g, feat, wcat, wself_t, wneigh_t, bvec, bout)


def _heads_body(parts_ref, muw_ref, mub_ref, sgw_ref, sgb_ref, mu_ref, sg_ref):
    x = jnp.sum(parts_ref[...], axis=0, keepdims=True) * (1.0 / N)
    mu_ref[...] = (
        jnp.dot(x, muw_ref[...], preferred_element_type=jnp.float32)
        + mub_ref[...]
    )
    sg_ref[...] = (
        jnp.dot(x, sgw_ref[...], preferred_element_type=jnp.float32)
        + sgb_ref[...]
    )


def _heads(parts, muw_t, mub, sgw_t, sgb):
    rep = muw_t.shape[1]
    return pl.pallas_call(
        _heads_body,
        out_shape=[
            jax.ShapeDtypeStruct((1, rep), jnp.float32),
            jax.ShapeDtypeStruct((1, rep), jnp.float32),
        ],
    )(parts, muw_t, mub, sgw_t, sgb)


def kernel(in_feat, edge_src, lstm1_Wih, lstm1_Whh, lstm1_bih, lstm1_bhh,
           fc_self1, fc_neigh1, bias1, lstm2_Wih, lstm2_Whh, lstm2_bih,
           lstm2_bhh, fc_self2, fc_neigh2, bias2, mu_W, mu_b, sigma_W, sigma_b):
    idxc = [edge_src[c * CN:(c + 1) * CN].T.reshape(-1) for c in range(NC)]
    featc = [in_feat[c * CN:(c + 1) * CN] for c in range(NC)]

    w1 = (jnp.concatenate([lstm1_Wih.T, lstm1_Whh.T], axis=0),
          fc_self1.T, fc_neigh1.T,
          (lstm1_bih + lstm1_bhh).reshape(1, GW), bias1.reshape(1, F))
    w2 = (jnp.concatenate([lstm2_Wih.T, lstm2_Whh.T], axis=0),
          fc_self2.T, fc_neigh2.T,
          (lstm2_bih + lstm2_bhh).reshape(1, GW), bias2.reshape(1, F))

    g1 = [None] * NC
    g1[0] = _sc_gather(in_feat, idxc[0])
    outs1 = []
    for c in range(NC):
        if c + 1 < NC:
            g1[c + 1] = _sc_gather(in_feat, idxc[c + 1])
        outs1.append(_layer1(g1[c].reshape(DEG, CN, F), featc[c], *w1))
    out1 = jnp.concatenate(outs1, axis=0)

    g2 = [None] * NC
    g2[0] = _sc_gather(out1, idxc[0])
    parts = []
    for c in range(NC):
        if c + 1 < NC:
            g2[c + 1] = _sc_gather(out1, idxc[c + 1])
        parts.append(_layer2(g2[c].reshape(DEG, CN, F),
                             outs1[c], *w2))
    mu, sigma = _heads(jnp.concatenate(parts, axis=0),
                       mu_W.T, mu_b.reshape(1, -1),
                       sigma_W.T, sigma_b.reshape(1, -1))
    return (mu, sigma)


# speedup vs baseline: 1.2613x; 1.0263x over previous
import functools

import jax
import jax.numpy as jnp
import numpy as np
from jax import lax
from jax.experimental import pallas as pl
from jax.experimental.pallas import tpu as pltpu
from jax.experimental.pallas import tpu_sc as plsc

N = 10000
DEG = 32
F = 128
GW = 4 * F

NC = 5
CN = N // NC
BC = CN * DEG

NB = 2000
CNT = CN // NB

NW = 32
RPW = BC // NW
CH = 400
NCHUNK = RPW // CH


def _sc_gather(table, idx_flat):
    mesh = plsc.VectorSubcoreMesh(core_axis_name="c", subcore_axis_name="s")

    @functools.partial(
        pl.kernel,
        mesh=mesh,
        out_type=jax.ShapeDtypeStruct((BC, F), jnp.float32),
        scratch_types=[
            pltpu.VMEM((CH,), jnp.int32),
            pltpu.VMEM((CH, F), jnp.float32),
            pltpu.SemaphoreType.DMA,
        ],
    )
    def gather_kernel(table_hbm, idx_hbm, out_hbm, idx_v, rows_v, sem):
        wid = lax.axis_index("s") * 2 + lax.axis_index("c")
        base = wid * RPW

        def body(k, carry):
            off = pl.multiple_of(base + k * CH, 8)
            pltpu.sync_copy(idx_hbm.at[pl.ds(off, CH)], idx_v)
            pltpu.async_copy(table_hbm.at[idx_v], rows_v, sem).wait()
            pltpu.sync_copy(rows_v, out_hbm.at[pl.ds(off, CH)])
            return carry

        lax.fori_loop(0, NCHUNK, body, 0)

    return gather_kernel(table, idx_flat)


def _lstm_step(xh_ref, c, wcat_ref, b_ref):
    gates = (
        jnp.dot(xh_ref[...], wcat_ref[...], preferred_element_type=jnp.float32)
        + b_ref[...]
    )
    tg = jnp.tanh(gates)
    ts = [tg[:, k * F:(k + 1) * F] for k in range(4)]
    i = 0.5 * ts[0] + 0.5
    f = 0.5 * ts[1] + 0.5
    g = ts[2]
    o = 0.5 * ts[3] + 0.5
    c2 = f * c + i * g
    h2 = o * jnp.tanh(c2)
    return h2, c2


def _layer1_body(g_ref, feat_ref, wcat_ref, wself_ref, wneigh_ref,
                 b_ref, bo_ref, out_ref, xh_ref, c_ref):
    t = pl.program_id(1)

    @pl.when(t == 0)
    def _():
        xh_ref[:, F:] = jnp.zeros((NB, F), jnp.float32)
        c_ref[...] = jnp.zeros_like(c_ref)

    xh_ref[:, :F] = g_ref[0]
    h2, c2 = _lstm_step(xh_ref, c_ref[...], wcat_ref, b_ref)
    xh_ref[:, F:] = h2
    c_ref[...] = c2

    @pl.when(t == DEG - 1)
    def _():
        out = (
            jnp.dot(feat_ref[...], wself_ref[...],
                    preferred_element_type=jnp.float32)
            + jnp.dot(h2, wneigh_ref[...],
                      preferred_element_type=jnp.float32)
            + bo_ref[...]
        )
        out_ref[...] = jnp.maximum(out, 0.0)


def _layer1(g, feat, wcat, wself_t, wneigh_t, bvec, bout):
    return pl.pallas_call(
        _layer1_body,
        grid=(CNT, DEG),
        in_specs=[
            pl.BlockSpec((1, NB, F), lambda i, t: (t, i, 0)),
            pl.BlockSpec((NB, F), lambda i, t: (i, 0)),
            pl.BlockSpec((2 * F, GW), lambda i, t: (0, 0)),
            pl.BlockSpec((F, F), lambda i, t: (0, 0)),
            pl.BlockSpec((F, F), lambda i, t: (0, 0)),
            pl.BlockSpec((1, GW), lambda i, t: (0, 0)),
            pl.BlockSpec((1, F), lambda i, t: (0, 0)),
        ],
        out_specs=pl.BlockSpec((NB, F), lambda i, t: (i, 0)),
        out_shape=jax.ShapeDtypeStruct((CN, F), jnp.float32),
        scratch_shapes=[
            pltpu.VMEM((NB, 2 * F), jnp.float32),
            pltpu.VMEM((NB, F), jnp.float32),
        ],
    )(g, feat, wcat, wself_t, wneigh_t, bvec, bout)


def _layer2_body(g_ref, feat_ref, wcat_ref, wself_ref, wneigh_ref,
                 b_ref, bo_ref, sum_ref, xh_ref, c_ref, acc_ref):
    i_ = pl.program_id(0)
    t = pl.program_id(1)

    @pl.when(jnp.logical_and(i_ == 0, t == 0))
    def _():
        acc_ref[...] = jnp.zeros_like(acc_ref)

    @pl.when(t == 0)
    def _():
        xh_ref[:, F:] = jnp.zeros((NB, F), jnp.float32)
        c_ref[...] = jnp.zeros_like(c_ref)

    xh_ref[:, :F] = g_ref[0]
    h2, c2 = _lstm_step(xh_ref, c_ref[...], wcat_ref, b_ref)
    xh_ref[:, F:] = h2
    c_ref[...] = c2

    @pl.when(t == DEG - 1)
    def _():
        out = (
            jnp.dot(feat_ref[...], wself_ref[...],
                    preferred_element_type=jnp.float32)
            + jnp.dot(h2, wneigh_ref[...],
                      preferred_element_type=jnp.float32)
            + bo_ref[...]
        )
        acc_ref[...] += jnp.sum(out, axis=0, keepdims=True)

    @pl.when(jnp.logical_and(i_ == CNT - 1, t == DEG - 1))
    def _():
        sum_ref[...] = acc_ref[...]


def _layer2(g, feat, wcat, wself_t, wneigh_t, bvec, bout):
    return pl.pallas_call(
        _layer2_body,
        grid=(CNT, DEG),
        in_specs=[
            pl.BlockSpec((1, NB, F), lambda i, t: (t, i, 0)),
            pl.BlockSpec((NB, F), lambda i, t: (i, 0)),
            pl.BlockSpec((2 * F, GW), lambda i, t: (0, 0)),
            pl.BlockSpec((F, F), lambda i, t: (0, 0)),
            pl.BlockSpec((F, F), lambda i, t: (0, 0)),
            pl.BlockSpec((1, GW), lambda i, t: (0, 0)),
            pl.BlockSpec((1, F), lambda i, t: (0, 0)),
        ],
        out_specs=pl.BlockSpec((1, F), lambda i, t: (0, 0)),
        out_shape=jax.ShapeDtypeStruct((1, F), jnp.float32),
        scratch_shapes=[
            pltpu.VMEM((NB, 2 * F), jnp.float32),
            pltpu.VMEM((NB, F), jnp.float32),
            pltpu.VMEM((1, F), jnp.float32),
        ],
    )(g, feat, wcat, wself_t, wneigh_t, bvec, bout)


def _heads_body(parts_ref, muw_ref, mub_ref, sgw_ref, sgb_ref, mu_ref, sg_ref):
    x = jnp.sum(parts_ref[...], axis=0, keepdims=True) * (1.0 / N)
    mu_ref[...] = (
        jnp.dot(x, muw_ref[...], preferred_element_type=jnp.float32)
        + mub_ref[...]
    )
    sg_ref[...] = (
        jnp.dot(x, sgw_ref[...], preferred_element_type=jnp.float32)
        + sgb_ref[...]
    )


def _heads(parts, muw_t, mub, sgw_t, sgb):
    rep = muw_t.shape[1]
    return pl.pallas_call(
        _heads_body,
        out_shape=[
            jax.ShapeDtypeStruct((1, rep), jnp.float32),
            jax.ShapeDtypeStruct((1, rep), jnp.float32),
        ],
    )(parts, muw_t, mub, sgw_t, sgb)


def kernel(in_feat, edge_src, lstm1_Wih, lstm1_Whh, lstm1_bih, lstm1_bhh,
           fc_self1, fc_neigh1, bias1, lstm2_Wih, lstm2_Whh, lstm2_bih,
           lstm2_bhh, fc_self2, fc_neigh2, bias2, mu_W, mu_b, sigma_W, sigma_b):
    idxc = [edge_src[c * CN:(c + 1) * CN].T.reshape(-1) for c in range(NC)]
    featc = [in_feat[c * CN:(c + 1) * CN] for c in range(NC)]

    gsc = jnp.concatenate([jnp.full((1, F), 0.5), jnp.full((1, F), 0.5),
                           jnp.ones((1, F)), jnp.full((1, F), 0.5)], axis=1)
    w1 = (jnp.concatenate([lstm1_Wih.T, lstm1_Whh.T], axis=0) * gsc,
          fc_self1.T, fc_neigh1.T,
          ((lstm1_bih + lstm1_bhh).reshape(1, GW)) * gsc, bias1.reshape(1, F))
    w2 = (jnp.concatenate([lstm2_Wih.T, lstm2_Whh.T], axis=0) * gsc,
          fc_self2.T, fc_neigh2.T,
          ((lstm2_bih + lstm2_bhh).reshape(1, GW)) * gsc, bias2.reshape(1, F))

    g1 = [None] * NC
    g1[0] = _sc_gather(in_feat, idxc[0])
    outs1 = []
    for c in range(NC):
        if c + 1 < NC:
            g1[c + 1] = _sc_gather(in_feat, idxc[c + 1])
        outs1.append(_layer1(g1[c].reshape(DEG, CN, F), featc[c], *w1))
    out1 = jnp.concatenate(outs1, axis=0)

    g2 = [None] * NC
    g2[0] = _sc_gather(out1, idxc[0])
    parts = []
    for c in range(NC):
        if c + 1 < NC:
            g2[c + 1] = _sc_gather(out1, idxc[c + 1])
        parts.append(_layer2(g2[c].reshape(DEG, CN, F),
                             outs1[c], *w2))
    mu, sigma = _heads(jnp.concatenate(parts, axis=0),
                       mu_W.T, mu_b.reshape(1, -1),
                       sigma_W.T, sigma_b.reshape(1, -1))
    return (mu, sigma)
